# Initial kernel scaffold; baseline (speedup 1.0000x reference)
import jax
import jax.numpy as jnp
from jax import lax
from jax.experimental import pallas as pl
from jax.experimental.pallas import tpu as pltpu
from jax.experimental.pallas import tpu_sc as plsc

N = 10000
E = 320000
F = 128
H = 128
O = 64

NC = 2
NS = 16
NW = NC * NS
K = 128
EW = E // NW
CH = -(-EW // K)
EWP = CH * K
E_PAD = NW * EWP
N_PAD = 10016
RT = N_PAD // NS

_MESH = plsc.VectorSubcoreMesh(
    core_axis_name="c", subcore_axis_name="s", num_cores=NC, num_subcores=NS)


def _deg_body(dst_hbm, zeros16_hbm, ones_hbm, out_hbm, idx_v, ones_v, acc):
    cid = lax.axis_index("c")
    sid = lax.axis_index("s")
    wid = sid * NC + cid
    pltpu.sync_copy(dst_hbm.at[wid], idx_v)
    pltpu.sync_copy(ones_hbm, ones_v)
    pltpu.sync_copy(zeros16_hbm.at[pl.ds(sid * RT, RT)],
                    acc.at[pl.ds(sid * RT, RT)])
    plsc.subcore_barrier()

    def body(j, carry):
        pltpu.sync_copy(ones_v, acc.at[idx_v.at[j]], add=True)
        return carry

    lax.fori_loop(0, CH, body, 0)
    plsc.subcore_barrier()
    pltpu.sync_copy(acc.at[pl.ds(sid * RT, RT)],
                    out_hbm.at[cid, pl.ds(sid * RT, RT)])


_deg_call = pl.kernel(
    _deg_body,
    out_type=jax.ShapeDtypeStruct((NC, N_PAD, 16), jnp.float32),
    mesh=_MESH,
    scratch_types=[
        pltpu.VMEM((CH, K), jnp.int32),
        pltpu.VMEM((K, 16), jnp.float32),
        pltpu.VMEM_SHARED((N_PAD, 16), jnp.float32),
    ],
)


def _make_seg_call(d):
    def body(y_hbm, src_hbm, dst_hbm, zeros_hbm, out_hbm, sidx, didx, rows, acc):
        cid = lax.axis_index("c")
        sid = lax.axis_index("s")
        wid = sid * NC + cid
        pltpu.sync_copy(src_hbm.at[wid], sidx)
        pltpu.sync_copy(dst_hbm.at[wid], didx)
        pltpu.sync_copy(zeros_hbm.at[pl.ds(sid * RT, RT)],
                        acc.at[pl.ds(sid * RT, RT)])
        plsc.subcore_barrier()

        def step(j, carry):
            pltpu.sync_copy(y_hbm.at[sidx.at[j]], rows)
            pltpu.sync_copy(rows, acc.at[didx.at[j]], add=True)
            return carry

        lax.fori_loop(0, CH, step, 0)
        plsc.subcore_barrier()
        pltpu.sync_copy(acc.at[pl.ds(sid * RT, RT)],
                        out_hbm.at[cid, pl.ds(sid * RT, RT)])

    return pl.kernel(
        body,
        out_type=jax.ShapeDtypeStruct((NC, N_PAD, d), jnp.float32),
        mesh=_MESH,
        scratch_types=[
            pltpu.VMEM((CH, K), jnp.int32),
            pltpu.VMEM((CH, K), jnp.int32),
            pltpu.VMEM((K, d), jnp.float32),
            pltpu.VMEM_SHARED((N_PAD, d), jnp.float32),
        ],
    )


_seg_call_H = _make_seg_call(H)
_seg_call_O = _make_seg_call(O)


def _tc1_body(x_ref, w_ref, degp_ref, y_ref, dinv_ref):
    deg = degp_ref[0][:, 0:1] + degp_ref[1][:, 0:1] + 1.0
    dinv = lax.rsqrt(deg)
    y_ref[...] = dinv * jnp.dot(x_ref[...], w_ref[...],
                                preferred_element_type=jnp.float32)
    dinv_ref[...] = dinv


_tc1_call = pl.pallas_call(
    _tc1_body,
    out_shape=(
        jax.ShapeDtypeStruct((N_PAD, H), jnp.float32),
        jax.ShapeDtypeStruct((N_PAD, 1), jnp.float32),
    ),
)


def _tc2_body(s_ref, y1_ref, dinv_ref, b1_ref, scale_ref, w2_ref, y2_ref):
    dinv = dinv_ref[...]
    h = dinv * (s_ref[0] + s_ref[1] + y1_ref[...]) + b1_ref[...]
    h = jnp.where(h >= 0.0, h, 0.01 * h)
    h = h * scale_ref[...]
    y2_ref[...] = dinv * jnp.dot(h, w2_ref[...],
                                 preferred_element_type=jnp.float32)


_tc2_call = pl.pallas_call(
    _tc2_body,
    out_shape=jax.ShapeDtypeStruct((N_PAD, O), jnp.float32),
)


def _tc3_body(s_ref, y2_ref, dinv_ref, b2_ref, out_ref):
    t = dinv_ref[...] * (s_ref[0] + s_ref[1] + y2_ref[...]) + b2_ref[...]
    out_ref[...] = jnp.where(t >= 0.0, t, 0.01 * t)


_tc3_call = pl.pallas_call(
    _tc3_body,
    out_shape=jax.ShapeDtypeStruct((N_PAD, O), jnp.float32),
)


def kernel(x, edge_idx, W1, b1, W2, b2):
    src = edge_idx[0]
    dst = edge_idx[1]
    padi = jnp.full((E_PAD - E,), N, jnp.int32)
    srcp = jnp.concatenate([src, padi]).reshape(NW, CH, K)
    dstp = jnp.concatenate([dst, padi]).reshape(NW, CH, K)
    xp = jnp.pad(x, ((0, N_PAD - N), (0, 0)))

    zeros16 = jnp.zeros((N_PAD, 16), jnp.float32)
    ones16 = jnp.ones((K, 16), jnp.float32)
    zerosH = jnp.zeros((N_PAD, H), jnp.float32)
    zerosO = jnp.zeros((N_PAD, O), jnp.float32)

    mask = jax.random.bernoulli(jax.random.key(42), 0.5, (N, H))
    scale = jnp.pad(jnp.where(mask, 2.0, 0.0).astype(jnp.float32),
                    ((0, N_PAD - N), (0, 0)))

    degp = _deg_call(dstp, zeros16, ones16)
    y1, dinv = _tc1_call(xp, W1, degp)
    s1 = _seg_call_H(y1, srcp, dstp, zerosH)
    y2 = _tc2_call(s1, y1, dinv, b1.reshape(1, H), scale, W2)
    s2 = _seg_call_O(y2, srcp, dstp, zerosO)
    out = _tc3_call(s2, y2, dinv, b2.reshape(1, O))
    return out[:N]


# speedup vs baseline: 15.8199x; 15.8199x over previous
import jax
import jax.numpy as jnp
from jax import lax
from jax.experimental import pallas as pl
from jax.experimental.pallas import tpu as pltpu
from jax.experimental.pallas import tpu_sc as plsc

N = 10000
E = 320000
F = 128
H = 128
O = 64

NC = 2
NS = 16
NW = NC * NS
K = 128
EW = E // NW
CH = -(-EW // K)
EWP = CH * K
E_PAD = NW * EWP
N_PAD = 10112
RT = N_PAD // NS

import functools


@functools.lru_cache(maxsize=None)
def _mesh():
    return plsc.VectorSubcoreMesh(
        core_axis_name="c", subcore_axis_name="s",
        num_cores=NC, num_subcores=NS)


def _deg_body(dst_hbm, zeros16_hbm, ones_hbm, out_hbm, idx_v, ones_v, acc):
    cid = lax.axis_index("c")
    sid = lax.axis_index("s")
    wid = sid * NC + cid
    pltpu.sync_copy(dst_hbm.at[wid], idx_v)
    pltpu.sync_copy(ones_hbm, ones_v)
    pltpu.sync_copy(zeros16_hbm.at[pl.ds(sid * RT, RT)],
                    acc.at[pl.ds(sid * RT, RT)])
    plsc.subcore_barrier()

    def body(j, carry):
        pltpu.sync_copy(ones_v, acc.at[idx_v.at[j]], add=True)
        return carry

    lax.fori_loop(0, CH, body, 0)
    plsc.subcore_barrier()
    pltpu.sync_copy(acc.at[pl.ds(sid * RT, RT)],
                    out_hbm.at[cid, pl.ds(sid * RT, RT)])


@functools.lru_cache(maxsize=None)
def _deg_call():
    return pl.kernel(
        _deg_body,
        out_type=jax.ShapeDtypeStruct((NC, N_PAD, 16), jnp.float32),
        mesh=_mesh(),
        compiler_params=pltpu.CompilerParams(use_tc_tiling_on_sc=False),
        scratch_types=[
            pltpu.VMEM((CH, K), jnp.int32),
            pltpu.VMEM((K, 16), jnp.float32),
            pltpu.VMEM_SHARED((N_PAD, 16), jnp.float32),
        ],
    )


def _make_seg_call(d):
    def body(y_hbm, src_hbm, dst_hbm, zeros_hbm, out_hbm, sidx, didx, rows, acc):
        cid = lax.axis_index("c")
        sid = lax.axis_index("s")
        wid = sid * NC + cid
        pltpu.sync_copy(src_hbm.at[wid], sidx)
        pltpu.sync_copy(dst_hbm.at[wid], didx)
        pltpu.sync_copy(zeros_hbm.at[pl.ds(sid * RT, RT)],
                        acc.at[pl.ds(sid * RT, RT)])
        plsc.subcore_barrier()

        def step(j, carry):
            pltpu.sync_copy(y_hbm.at[sidx.at[j]], rows)
            pltpu.sync_copy(rows, acc.at[didx.at[j]], add=True)
            return carry

        lax.fori_loop(0, CH, step, 0)
        plsc.subcore_barrier()
        pltpu.sync_copy(acc.at[pl.ds(sid * RT, RT)],
                        out_hbm.at[cid, pl.ds(sid * RT, RT)])

    return pl.kernel(
        body,
        out_type=jax.ShapeDtypeStruct((NC, N_PAD, d), jnp.float32),
        mesh=_mesh(),
        compiler_params=pltpu.CompilerParams(use_tc_tiling_on_sc=False),
        scratch_types=[
            pltpu.VMEM((CH, K), jnp.int32),
            pltpu.VMEM((CH, K), jnp.int32),
            pltpu.VMEM((K, d), jnp.float32),
            pltpu.VMEM_SHARED((N_PAD, d), jnp.float32),
        ],
    )


_make_seg_call = functools.lru_cache(maxsize=None)(_make_seg_call)


def _tc1_body(x_ref, w_ref, degp_ref, y_ref, dinv_ref):
    deg = degp_ref[0][:, 0:1] + degp_ref[1][:, 0:1] + 1.0
    dinv = lax.rsqrt(deg)
    y_ref[...] = dinv * jnp.dot(x_ref[...], w_ref[...],
                                preferred_element_type=jnp.float32)
    dinv_ref[...] = dinv


_tc1_call = pl.pallas_call(
    _tc1_body,
    out_shape=(
        jax.ShapeDtypeStruct((N_PAD, H), jnp.float32),
        jax.ShapeDtypeStruct((N_PAD, 1), jnp.float32),
    ),
)


def _tc2_body(s_ref, y1_ref, dinv_ref, b1_ref, scale_ref, w2_ref, y2_ref):
    dinv = dinv_ref[...]
    h = dinv * (s_ref[0] + s_ref[1] + y1_ref[...]) + b1_ref[...]
    h = jnp.where(h >= 0.0, h, 0.01 * h)
    h = h * scale_ref[...]
    y2_ref[...] = dinv * jnp.dot(h, w2_ref[...],
                                 preferred_element_type=jnp.float32)


_tc2_call = pl.pallas_call(
    _tc2_body,
    out_shape=jax.ShapeDtypeStruct((N_PAD, O), jnp.float32),
)


def _tc3_body(s_ref, y2_ref, dinv_ref, b2_ref, out_ref):
    t = dinv_ref[...] * (s_ref[0] + s_ref[1] + y2_ref[...]) + b2_ref[...]
    out_ref[...] = jnp.where(t >= 0.0, t, 0.01 * t)


_tc3_call = pl.pallas_call(
    _tc3_body,
    out_shape=jax.ShapeDtypeStruct((N_PAD, O), jnp.float32),
)


def kernel(x, edge_idx, W1, b1, W2, b2):
    src = edge_idx[0]
    dst = edge_idx[1]
    padi = jnp.full((E_PAD - E,), N, jnp.int32)
    srcp = jnp.concatenate([src, padi]).reshape(NW, CH, K)
    dstp = jnp.concatenate([dst, padi]).reshape(NW, CH, K)
    xp = jnp.pad(x, ((0, N_PAD - N), (0, 0)))

    zeros16 = jnp.zeros((N_PAD, 16), jnp.float32)
    ones16 = jnp.ones((K, 16), jnp.float32)
    zerosH = jnp.zeros((N_PAD, H), jnp.float32)
    zerosO = jnp.zeros((N_PAD, O), jnp.float32)

    mask = jax.random.bernoulli(jax.random.key(42), 0.5, (N, H))
    scale = jnp.pad(jnp.where(mask, 2.0, 0.0).astype(jnp.float32),
                    ((0, N_PAD - N), (0, 0)))

    degp = _deg_call()(dstp, zeros16, ones16)
    y1, dinv = _tc1_call(xp, W1, degp)
    s1 = _make_seg_call(H)(y1, srcp, dstp, zerosH)
    y2 = _tc2_call(s1, y1, dinv, b1.reshape(1, H), scale, W2)
    s2 = _make_seg_call(O)(y2, srcp, dstp, zerosO)
    out = _tc3_call(s2, y2, dinv, b2.reshape(1, O))
    return out[:N]
